# Initial kernel scaffold; baseline (speedup 1.0000x reference)
import jax
import jax.numpy as jnp
from jax.experimental import pallas as pl


def kernel(x, A, mask, k):
    raise NotImplementedError("write your pallas kernel here")


# speedup vs baseline: 20.9025x; 20.9025x over previous
import jax
import jax.numpy as jnp
from jax import lax
from jax.experimental import pallas as pl
from jax.experimental.pallas import tpu as pltpu

_B, _C, _N, _D, _L = 4, 4, 1000, 64, 12
_RB = 200
_NRB = _N // _RB
_BETA = 0.02
_INV_SQRT10 = 10.0 ** -0.5
_K = 5


def _fused_kernel(x_ref, a_ref, m_ref, am_ref, mu_ref):
    rb = pl.program_id(1)
    row0 = rb * _RB

    s_sum = jnp.zeros((_RB, _N), jnp.float32)
    for c in range(_C):
        xr = x_ref[0, c, pl.ds(row0, _RB), :]
        xc = x_ref[0, c]
        logits = lax.dot_general(xr, xc, (((1,), (1,)), ((), ())),
                                 preferred_element_type=jnp.float32)
        mx = jnp.max(logits, axis=-1, keepdims=True)
        e = jnp.exp(logits - mx)
        s_sum = s_sum + e / jnp.sum(e, axis=-1, keepdims=True)
    s_mean = s_sum * (1.0 / _C)

    a_mean = (a_ref[0, 0] + a_ref[0, 1] + a_ref[0, 2] + a_ref[0, 3]) * (1.0 / _C)

    iota = lax.broadcasted_iota(jnp.int32, (_RB, _N), 1)
    for c in range(_C):
        mc = m_ref[0, c]
        mr = m_ref[0, c, pl.ds(row0, _RB), :]
        prod = lax.dot_general(mr, mc, (((1,), (1,)), ((), ())),
                               preferred_element_type=jnp.float32)
        mw = jax.nn.sigmoid(s_mean * prod * _INV_SQRT10)
        am_c = a_mean + _BETA * mw
        am_ref[0, c] = am_c

        work = am_c
        best = jnp.full((_RB, _L), -jnp.inf, jnp.float32)
        for _ in range(_K):
            rmax = jnp.max(work, axis=-1, keepdims=True)
            ismax = work == rmax
            idx = jnp.min(jnp.where(ismax, iota, _N), axis=-1, keepdims=True)
            sel = iota == idx
            row = lax.dot_general(sel.astype(jnp.float32), mc,
                                  (((1,), (0,)), ((), ())),
                                  preferred_element_type=jnp.float32)
            best = jnp.maximum(best, row)
            work = jnp.where(sel, -jnp.inf, work)
        mu_ref[0, c] = best


def kernel(x, A, mask, k):
    del k
    grid = (_B, _NRB)
    am, mu = pl.pallas_call(
        _fused_kernel,
        grid=grid,
        in_specs=[
            pl.BlockSpec((1, _C, _N, _D), lambda b, r: (b, 0, 0, 0)),
            pl.BlockSpec((1, _C, _RB, _N), lambda b, r: (b, 0, r, 0)),
            pl.BlockSpec((1, _C, _N, _L), lambda b, r: (b, 0, 0, 0)),
        ],
        out_specs=[
            pl.BlockSpec((1, _C, _RB, _N), lambda b, r: (b, 0, r, 0)),
            pl.BlockSpec((1, _C, _RB, _L), lambda b, r: (b, 0, r, 0)),
        ],
        out_shape=[
            jax.ShapeDtypeStruct((_B, _C, _N, _N), jnp.float32),
            jax.ShapeDtypeStruct((_B, _C, _N, _L), jnp.float32),
        ],
        compiler_params=pltpu.CompilerParams(
            dimension_semantics=("parallel", "parallel"),
        ),
    )(x, A, mask)
    return (am, mu)


# speedup vs baseline: 30.0653x; 1.4384x over previous
import jax
import jax.numpy as jnp
from jax import lax
from jax.experimental import pallas as pl
from jax.experimental.pallas import tpu as pltpu

_B, _C, _N, _D, _L = 4, 4, 1000, 64, 12
_RB = 200
_NRB = _N // _RB
_BETA = 0.02
_INV_SQRT10 = 10.0 ** -0.5
_K = 5


def _fused_kernel(x_ref, a_ref, m_ref, am_ref, mu_ref):
    rb = pl.program_id(1)
    row0 = rb * _RB

    s_sum = jnp.zeros((_RB, _N), jnp.float32)
    for c in range(_C):
        xr = x_ref[0, c, pl.ds(row0, _RB), :].astype(jnp.bfloat16)
        xc = x_ref[0, c].astype(jnp.bfloat16)
        logits = lax.dot_general(xr, xc, (((1,), (1,)), ((), ())),
                                 preferred_element_type=jnp.float32)
        mx = jnp.max(logits, axis=-1, keepdims=True)
        e = jnp.exp(logits - mx)
        s_sum = s_sum + e / jnp.sum(e, axis=-1, keepdims=True)
    s_mean = s_sum * (1.0 / _C)

    a_mean = (a_ref[0, 0] + a_ref[0, 1] + a_ref[0, 2] + a_ref[0, 3]) * (1.0 / _C)

    rev_iota = 999 - lax.broadcasted_iota(jnp.int32, (_RB, _N), 1)
    for c in range(_C):
        mc = m_ref[0, c]
        mr = m_ref[0, c, pl.ds(row0, _RB), :]
        prod = lax.dot_general(mr.astype(jnp.bfloat16), mc.astype(jnp.bfloat16),
                               (((1,), (1,)), ((), ())),
                               preferred_element_type=jnp.float32)
        mw = jax.nn.sigmoid(s_mean * prod * _INV_SQRT10)
        am_c = a_mean + _BETA * mw
        am_ref[0, c] = am_c

        bits = lax.bitcast_convert_type(am_c, jnp.int32)
        keys = lax.bitcast_convert_type(((bits + 512) & ~1023) | rev_iota,
                                        jnp.float32)
        best = jnp.full((_RB, _L), -jnp.inf, jnp.float32)
        for j in range(_K):
            kmax = jnp.max(keys, axis=-1, keepdims=True)
            sel = keys == kmax
            row = lax.dot_general(sel.astype(jnp.float32), mc,
                                  (((1,), (0,)), ((), ())),
                                  preferred_element_type=jnp.float32)
            best = jnp.maximum(best, row)
            if j < _K - 1:
                keys = jnp.where(sel, -jnp.inf, keys)
        mu_ref[0, c] = best


def kernel(x, A, mask, k):
    del k
    grid = (_B, _NRB)
    am, mu = pl.pallas_call(
        _fused_kernel,
        grid=grid,
        in_specs=[
            pl.BlockSpec((1, _C, _N, _D), lambda b, r: (b, 0, 0, 0)),
            pl.BlockSpec((1, _C, _RB, _N), lambda b, r: (b, 0, r, 0)),
            pl.BlockSpec((1, _C, _N, _L), lambda b, r: (b, 0, 0, 0)),
        ],
        out_specs=[
            pl.BlockSpec((1, _C, _RB, _N), lambda b, r: (b, 0, r, 0)),
            pl.BlockSpec((1, _C, _RB, _L), lambda b, r: (b, 0, r, 0)),
        ],
        out_shape=[
            jax.ShapeDtypeStruct((_B, _C, _N, _N), jnp.float32),
            jax.ShapeDtypeStruct((_B, _C, _N, _L), jnp.float32),
        ],
        compiler_params=pltpu.CompilerParams(
            dimension_semantics=("parallel", "parallel"),
        ),
    )(x, A, mask)
    return (am, mu)


# speedup vs baseline: 31.3619x; 1.0431x over previous
import jax
import jax.numpy as jnp
from jax import lax
from jax.experimental import pallas as pl
from jax.experimental.pallas import tpu as pltpu

_B, _C, _N, _D, _L = 4, 4, 1000, 64, 12
_RB = 200
_NRB = _N // _RB
_BETA = 0.02
_INV_SQRT10 = 10.0 ** -0.5
_K = 5


def _fused_kernel(x_ref, a_ref, m_ref, am_ref, mu_ref):
    rb = pl.program_id(1)
    row0 = rb * _RB

    s_sum = jnp.zeros((_RB, _N), jnp.float32)
    for c in range(_C):
        xr = x_ref[0, c, pl.ds(row0, _RB), :].astype(jnp.bfloat16)
        xc = x_ref[0, c].astype(jnp.bfloat16)
        logits = lax.dot_general(xr, xc, (((1,), (1,)), ((), ())),
                                 preferred_element_type=jnp.float32)
        mx = jnp.max(logits, axis=-1, keepdims=True)
        e = jnp.exp(logits - mx)
        s_sum = s_sum + e / jnp.sum(e, axis=-1, keepdims=True)

    a_base = (a_ref[0, 0] + a_ref[0, 1] + a_ref[0, 2] + a_ref[0, 3]) * (1.0 / _C) \
        + (_BETA / 2)
    z_scale = 0.5 * _INV_SQRT10 / _C

    rev_iota = 999 - lax.broadcasted_iota(jnp.int32, (_RB, _N), 1)
    for c in range(_C):
        mc = m_ref[0, c]
        mr = m_ref[0, c, pl.ds(row0, _RB), :]
        prod = lax.dot_general(mr.astype(jnp.bfloat16), mc.astype(jnp.bfloat16),
                               (((1,), (1,)), ((), ())),
                               preferred_element_type=jnp.float32)
        am_c = a_base + (_BETA / 2) * jnp.tanh(s_sum * prod * z_scale)
        am_ref[0, c] = am_c

        bits = lax.bitcast_convert_type(am_c, jnp.int32)
        keys = lax.bitcast_convert_type(((bits + 512) & ~1023) | rev_iota,
                                        jnp.float32)
        best = jnp.full((_RB, _L), -jnp.inf, jnp.float32)
        for j in range(_K):
            kmax = jnp.max(keys, axis=-1, keepdims=True)
            sel = keys == kmax
            row = lax.dot_general(sel.astype(jnp.float32), mc,
                                  (((1,), (0,)), ((), ())),
                                  preferred_element_type=jnp.float32)
            best = jnp.maximum(best, row)
            if j < _K - 1:
                keys = jnp.where(sel, -jnp.inf, keys)
        mu_ref[0, c] = best


def kernel(x, A, mask, k):
    del k
    grid = (_B, _NRB)
    am, mu = pl.pallas_call(
        _fused_kernel,
        grid=grid,
        in_specs=[
            pl.BlockSpec((1, _C, _N, _D), lambda b, r: (b, 0, 0, 0)),
            pl.BlockSpec((1, _C, _RB, _N), lambda b, r: (b, 0, r, 0)),
            pl.BlockSpec((1, _C, _N, _L), lambda b, r: (b, 0, 0, 0)),
        ],
        out_specs=[
            pl.BlockSpec((1, _C, _RB, _N), lambda b, r: (b, 0, r, 0)),
            pl.BlockSpec((1, _C, _RB, _L), lambda b, r: (b, 0, r, 0)),
        ],
        out_shape=[
            jax.ShapeDtypeStruct((_B, _C, _N, _N), jnp.float32),
            jax.ShapeDtypeStruct((_B, _C, _N, _L), jnp.float32),
        ],
        compiler_params=pltpu.CompilerParams(
            dimension_semantics=("parallel", "parallel"),
        ),
    )(x, A, mask)
    return (am, mu)
